# baseline (speedup 1.0000x reference)
import functools

import jax
import jax.numpy as jnp
from jax import lax
from jax.experimental import pallas as pl
from jax.experimental.pallas import tpu as pltpu
from jax.experimental.pallas import tpu_sc as plsc

N = 10000
E = 320000
D = 128
G = 64

NW = 32
EPW = 10112
NGRP = EPW // 16
THIRD = 3456
SROWS = THIRD // 16
CHUNK = 64
BCH = 8
NBK = 8
CAP = NBK * BCH * CHUNK
CAPV = CAP + 256
N_ACC = 10112
RPT = N_ACC // 16
DUMMY = N
BR = 1000

_mesh = plsc.VectorSubcoreMesh(core_axis_name="c", subcore_axis_name="s")


@functools.partial(
    pl.kernel,
    mesh=_mesh,
    out_type=jax.ShapeDtypeStruct((NW * 3, CAP), jnp.int32),
    scratch_types=[
        pltpu.VMEM((EPW,), jnp.int32),
        pltpu.VMEM((EPW,), jnp.int32),
        pltpu.VMEM((128,), jnp.int32),
        pltpu.VMEM((128,), jnp.int32),
        pltpu.VMEM((48,), jnp.int32),
        pltpu.VMEM_SHARED((16 * 3 * CAPV,), jnp.int32),
    ],
)
def _sc_partition(src_hbm, dst_hbm, led_hbm, src_v, dst_v, vals_v, tgt_v,
                  offs_v, bkt_sh):
    cid = lax.axis_index("c")
    sid = lax.axis_index("s")
    wid = cid * 16 + sid

    pltpu.sync_copy(src_hbm.at[wid], src_v)
    pltpu.sync_copy(dst_hbm.at[wid], dst_v)

    dvp = jnp.full((16,), DUMMY << 16, jnp.int32)

    def fill(m, c):
        for jj in range(8):
            vals_v[pl.ds(jj * 16, 16)] = dvp
        pltpu.sync_copy(vals_v,
                        bkt_sh.at[pl.ds(sid * 3 * CAPV + m * 128, 128)])
        return c

    lax.fori_loop(0, 3 * CAPV // 128, fill, 0)

    th1 = jnp.full((16,), THIRD, jnp.int32)
    th2 = jnp.full((16,), 2 * THIRD, jnp.int32)
    one = jnp.full((16,), 1, jnp.int32)
    zero = jnp.zeros((16,), jnp.int32)
    sh16 = jnp.full((16,), 16, jnp.int32)

    def ge01(a, b):
        return jnp.minimum(jnp.maximum(a - b + one, zero), one)

    iota = lax.iota(jnp.int32, 16)

    dn = lax.GatherDimensionNumbers(offset_dims=(), collapsed_slice_dims=(0,),
                                    start_index_map=(0,))

    def lgather(x, idx):
        return lax.gather(x, idx[:, None], dn, slice_sizes=(1,),
                          mode=lax.GatherScatterMode.PROMISE_IN_BOUNDS)

    def prefix(ck):
        x = ck
        for d in (1, 2, 4, 8):
            dv = jnp.full((16,), d, jnp.int32)
            m = ge01(iota, dv)
            x = x + m * lgather(x, jnp.maximum(iota - dv, zero))
        return x

    lane15 = jnp.full((16,), 15, jnp.int32)

    for k in range(3):
        offs_v[pl.ds(k * 16, 16)] = zero

    def body(g, c):
        s16 = src_v[pl.ds(g * 16, 16)]
        d16 = dst_v[pl.ds(g * 16, 16)]
        t16 = ge01(s16, th1) + ge01(s16, th2)
        loc = s16 - t16 * th1
        tgt = zero
        pks, cks = [], []
        for k in range(2):
            dk = t16 - jnp.full((16,), k, jnp.int32)
            ck = one - jnp.minimum(dk * dk, one)
            cks.append(ck)
            pks.append(prefix(ck))
        cks.append(one - cks[0] - cks[1])
        pks.append(iota + one - pks[0] - pks[1])
        for k in range(3):
            off_k = offs_v[pl.ds(k * 16, 16)]
            ck, pk = cks[k], pks[k]
            tgt = tgt + ck * (jnp.full((16,), k * CAPV, jnp.int32)
                              + off_k + pk - one)
            offs_v[pl.ds(k * 16, 16)] = off_k + lgather(pk, lane15)
        gm = g % 8
        vals_v[pl.ds(gm * 16, 16)] = loc + jnp.left_shift(d16, sh16)
        tgt_v[pl.ds(gm * 16, 16)] = tgt + sid * (3 * CAPV)

        @pl.when(gm == 7)
        def _():
            pltpu.sync_copy(vals_v, bkt_sh.at[tgt_v])

        return c

    lax.fori_loop(0, NGRP, body, 0)

    for k in range(3):
        pltpu.sync_copy(bkt_sh.at[pl.ds(sid * 3 * CAPV + k * CAPV, CAP)],
                        led_hbm.at[wid * 3 + k])


@functools.partial(
    pl.kernel,
    mesh=_mesh,
    out_type=jax.ShapeDtypeStruct((2, N_ACC, 128), jnp.float32),
    scratch_types=[
        pltpu.VMEM((BCH, CHUNK), jnp.int32),
        pltpu.VMEM((BCH, CHUNK), jnp.int32),
        pltpu.VMEM((BCH, CHUNK), jnp.int32),
        pltpu.VMEM((2, CHUNK, 128), jnp.float32),
        pltpu.VMEM_SHARED((THIRD, 128), jnp.float32),
        pltpu.VMEM_SHARED((N_ACC, 128), jnp.float32),
        pltpu.SemaphoreType.DMA,
        pltpu.SemaphoreType.DMA,
        pltpu.SemaphoreType.DMA,
        pltpu.SemaphoreType.DMA,
    ],
)
def _sc_layer(h_hbm, led_hbm, agg_hbm, src_v, dst_v, pk_v, rows_v,
              h_sh, acc_sh, g0, g1, s0, s1):
    gsems = (g0, g1)
    ssems = (s0, s1)
    cid = lax.axis_index("c")
    sid = lax.axis_index("s")
    wid = cid * 16 + sid

    zv = jnp.zeros((16,), jnp.float32)

    def zbody(i, c):
        for jj in range(8):
            rows_v[0, i, pl.ds(jj * 16, 16)] = zv
        return c

    lax.fori_loop(0, CHUNK, zbody, 0)
    base = sid * RPT
    for k in range(RPT // CHUNK):
        pltpu.sync_copy(rows_v.at[0], acc_sh.at[pl.ds(base + k * CHUNK, CHUNK)])
    rem = RPT % CHUNK
    if rem:
        pltpu.sync_copy(rows_v.at[0, pl.ds(0, rem)],
                        acc_sh.at[pl.ds(base + RPT - rem, rem)])

    def g_start(u, buf):
        pltpu.make_async_copy(h_sh.at[src_v.at[u]], rows_v.at[buf],
                              gsems[buf]).start()

    def g_wait(u, buf):
        pltpu.make_async_copy(h_sh.at[src_v.at[u]], rows_v.at[buf],
                              gsems[buf]).wait()

    def s_start(u, buf):
        pltpu.async_copy(rows_v.at[buf], acc_sh.at[dst_v.at[u]],
                         ssems[buf], add=True)

    def s_wait(u, buf):
        pltpu.make_async_copy(rows_v.at[buf], acc_sh.at[dst_v.at[u]],
                              ssems[buf]).wait()

    for p in range(3):
        soff = pl.multiple_of(p * THIRD + sid * SROWS, 8)
        if p < 2:
            pltpu.sync_copy(h_hbm.at[pl.ds(soff, SROWS)],
                            h_sh.at[pl.ds(sid * SROWS, SROWS)])
        else:
            @pl.when(sid <= 13)
            def _():
                pltpu.sync_copy(h_hbm.at[pl.ds(soff, SROWS)],
                                h_sh.at[pl.ds(sid * SROWS, SROWS)])

            @pl.when(sid == 14)
            def _():
                pltpu.sync_copy(
                    h_hbm.at[pl.ds(pl.multiple_of(2 * THIRD + 14 * SROWS, 8),
                                   N - 2 * THIRD - 14 * SROWS)],
                    h_sh.at[pl.ds(14 * SROWS, N - 2 * THIRD - 14 * SROWS)])

        plsc.subcore_barrier()

        mask16 = jnp.full((16,), 0xFFFF, jnp.int32)
        sh16 = jnp.full((16,), 16, jnp.int32)

        def blk(b, c):
            boff = pl.multiple_of(b * BCH, BCH)
            pltpu.sync_copy(led_hbm.at[wid * 3 + p, pl.ds(boff, BCH)], pk_v)
            for r in range(BCH):
                for q in range(CHUNK // 16):
                    w = pk_v[r, pl.ds(q * 16, 16)]
                    src_v[r, pl.ds(q * 16, 16)] = jnp.bitwise_and(w, mask16)
                    dst_v[r, pl.ds(q * 16, 16)] = lax.shift_right_logical(
                        w, sh16)
            g_start(0, 0)
            g_start(1, 1)
            for u in range(BCH):
                g_wait(u, u % 2)
                s_start(u, u % 2)
                s_wait(u, u % 2)
                if u + 2 < BCH:
                    g_start(u + 2, u % 2)
            return c

        lax.fori_loop(0, NBK, blk, 0)
        plsc.subcore_barrier()

    pltpu.sync_copy(acc_sh.at[pl.ds(base, RPT)],
                    agg_hbm.at[cid, pl.ds(base, RPT)])


def _mlp_body(h_ref, a_ref, batch_ref, w1_ref, b1_ref, w2_ref, b2_ref,
              out_ref, pool_ref):
    a = h_ref[...] + a_ref[0] + a_ref[1]
    t = jnp.maximum(
        jnp.dot(a, w1_ref[...], preferred_element_type=jnp.float32)
        + b1_ref[...], 0.0)
    o = jnp.maximum(
        jnp.dot(t, w2_ref[...], preferred_element_type=jnp.float32)
        + b2_ref[...], 0.0)
    out_ref[...] = o
    bt = jnp.reshape(batch_ref[...], (1, BR))
    onehot = (lax.broadcasted_iota(jnp.int32, (G, BR), 0) == bt
              ).astype(jnp.float32)
    contrib = jnp.dot(onehot, o, preferred_element_type=jnp.float32)

    @pl.when(pl.program_id(0) == 0)
    def _():
        pool_ref[...] = contrib

    @pl.when(pl.program_id(0) != 0)
    def _():
        pool_ref[...] += contrib


_mlp = pl.pallas_call(
    _mlp_body,
    grid=(N // BR,),
    in_specs=[
        pl.BlockSpec((BR, 128), lambda i: (i, 0)),
        pl.BlockSpec((2, BR, 128), lambda i: (0, i, 0)),
        pl.BlockSpec((1, 1, BR), lambda i: (i, 0, 0)),
        pl.BlockSpec((128, 128), lambda i: (0, 0)),
        pl.BlockSpec((1, 128), lambda i: (0, 0)),
        pl.BlockSpec((128, 128), lambda i: (0, 0)),
        pl.BlockSpec((1, 128), lambda i: (0, 0)),
    ],
    out_specs=[
        pl.BlockSpec((BR, 128), lambda i: (i, 0)),
        pl.BlockSpec((G, 128), lambda i: (0, 0)),
    ],
    out_shape=[
        jax.ShapeDtypeStruct((N, 128), jnp.float32),
        jax.ShapeDtypeStruct((G, 128), jnp.float32),
    ],
    compiler_params=pltpu.CompilerParams(
        dimension_semantics=("arbitrary",)),
)


def kernel(x, edge_index, batch, W1_0, b1_0, W2_0, b2_0, W1_1, b1_1, W2_1,
           b2_1, W1_2, b1_2, W2_2, b2_2):
    src = edge_index[0].astype(jnp.int32)
    dst = edge_index[1].astype(jnp.int32)
    pad = NW * EPW - E
    pad_src = (jnp.arange(pad, dtype=jnp.int32) * N) // pad
    src_p = jnp.concatenate([src, pad_src]).reshape(NW, EPW)
    dst_p = jnp.concatenate([dst, jnp.full((pad,), DUMMY, jnp.int32)]
                            ).reshape(NW, EPW)
    batch3 = batch.astype(jnp.int32).reshape(N // BR, 1, BR)

    led = _sc_partition(src_p, dst_p)
    led4 = led.reshape(NW * 3, NBK * BCH, CHUNK)

    layers = [(W1_0, b1_0, W2_0, b2_0), (W1_1, b1_1, W2_1, b2_1),
              (W1_2, b1_2, W2_2, b2_2)]
    h = x
    hs, pools = [], []
    for (W1, b1, W2, b2) in layers:
        agg = _sc_layer(h, led4)
        h, pool = _mlp(h, agg, batch3, W1, b1.reshape(1, 128), W2,
                       b2.reshape(1, 128))
        hs.append(h)
        pools.append(pool)
    graph_emb = jnp.concatenate(pools, axis=1)
    node_emb = jnp.concatenate(hs, axis=1)
    return (graph_emb, node_emb)


# speedup vs baseline: 1.1165x; 1.1165x over previous
import functools

import jax
import jax.numpy as jnp
from jax import lax
from jax.experimental import pallas as pl
from jax.experimental.pallas import tpu as pltpu
from jax.experimental.pallas import tpu_sc as plsc

N = 10000
E = 320000
D = 128
G = 64

NW = 32
EPW = 10112
NGRP = EPW // 16
THIRD = 3456
SROWS = THIRD // 16
CHUNK = 64
BCH = 12
NBK = 5
CAP = NBK * BCH * CHUNK
CAPV = CAP + 256
N_ACC = 10112
RPT = N_ACC // 16
DUMMY = N
BR = 1000

_mesh = plsc.VectorSubcoreMesh(core_axis_name="c", subcore_axis_name="s")


@functools.partial(
    pl.kernel,
    mesh=_mesh,
    out_type=jax.ShapeDtypeStruct((NW * 3, 1, CAP), jnp.int32),
    scratch_types=[
        pltpu.VMEM((EPW,), jnp.int32),
        pltpu.VMEM((EPW,), jnp.int32),
        pltpu.VMEM((128,), jnp.int32),
        pltpu.VMEM((128,), jnp.int32),
        pltpu.VMEM((48,), jnp.int32),
        pltpu.VMEM_SHARED((16 * 3 * CAPV,), jnp.int32),
    ],
)
def _sc_partition(src_hbm, dst_hbm, led_hbm, src_v, dst_v, vals_v, tgt_v,
                  offs_v, bkt_sh):
    cid = lax.axis_index("c")
    sid = lax.axis_index("s")
    wid = cid * 16 + sid

    pltpu.sync_copy(src_hbm.at[wid], src_v)
    pltpu.sync_copy(dst_hbm.at[wid], dst_v)

    dvp = jnp.full((16,), DUMMY << 16, jnp.int32)

    def fill(m, c):
        for jj in range(8):
            vals_v[pl.ds(jj * 16, 16)] = dvp
        pltpu.sync_copy(vals_v,
                        bkt_sh.at[pl.ds(sid * 3 * CAPV + m * 128, 128)])
        return c

    lax.fori_loop(0, 3 * CAPV // 128, fill, 0)

    th1 = jnp.full((16,), THIRD, jnp.int32)
    th2 = jnp.full((16,), 2 * THIRD, jnp.int32)
    one = jnp.full((16,), 1, jnp.int32)
    zero = jnp.zeros((16,), jnp.int32)
    sh16 = jnp.full((16,), 16, jnp.int32)

    def ge01(a, b):
        return jnp.minimum(jnp.maximum(a - b + one, zero), one)

    iota = lax.iota(jnp.int32, 16)

    dn = lax.GatherDimensionNumbers(offset_dims=(), collapsed_slice_dims=(0,),
                                    start_index_map=(0,))

    def lgather(x, idx):
        return lax.gather(x, idx[:, None], dn, slice_sizes=(1,),
                          mode=lax.GatherScatterMode.PROMISE_IN_BOUNDS)

    def prefix(ck):
        x = ck
        for d in (1, 2, 4, 8):
            dv = jnp.full((16,), d, jnp.int32)
            m = ge01(iota, dv)
            x = x + m * lgather(x, jnp.maximum(iota - dv, zero))
        return x

    lane15 = jnp.full((16,), 15, jnp.int32)

    for k in range(3):
        offs_v[pl.ds(k * 16, 16)] = zero

    def body(g, c):
        s16 = src_v[pl.ds(g * 16, 16)]
        d16 = dst_v[pl.ds(g * 16, 16)]
        t16 = ge01(s16, th1) + ge01(s16, th2)
        loc = s16 - t16 * th1
        tgt = zero
        pks, cks = [], []
        for k in range(2):
            dk = t16 - jnp.full((16,), k, jnp.int32)
            ck = one - jnp.minimum(dk * dk, one)
            cks.append(ck)
            pks.append(prefix(ck))
        cks.append(one - cks[0] - cks[1])
        pks.append(iota + one - pks[0] - pks[1])
        for k in range(3):
            off_k = offs_v[pl.ds(k * 16, 16)]
            ck, pk = cks[k], pks[k]
            tgt = tgt + ck * (jnp.full((16,), k * CAPV, jnp.int32)
                              + off_k + pk - one)
            offs_v[pl.ds(k * 16, 16)] = off_k + lgather(pk, lane15)
        gm = g % 8
        vals_v[pl.ds(gm * 16, 16)] = loc + jnp.left_shift(d16, sh16)
        tgt_v[pl.ds(gm * 16, 16)] = tgt + sid * (3 * CAPV)

        @pl.when(gm == 7)
        def _():
            pltpu.sync_copy(vals_v, bkt_sh.at[tgt_v])

        return c

    lax.fori_loop(0, NGRP, body, 0)

    for k in range(3):
        pltpu.sync_copy(bkt_sh.at[pl.ds(sid * 3 * CAPV + k * CAPV, CAP)],
                        led_hbm.at[wid * 3 + k, 0])


@functools.partial(
    pl.kernel,
    mesh=_mesh,
    out_type=jax.ShapeDtypeStruct((2, N_ACC, 128), jnp.float32),
    scratch_types=[
        pltpu.VMEM((BCH, CHUNK), jnp.int32),
        pltpu.VMEM((BCH, CHUNK), jnp.int32),
        pltpu.VMEM((BCH * CHUNK,), jnp.int32),
        pltpu.VMEM((2, CHUNK, 128), jnp.float32),
        pltpu.VMEM_SHARED((THIRD, 128), jnp.float32),
        pltpu.VMEM_SHARED((N_ACC, 128), jnp.float32),
        pltpu.SemaphoreType.DMA,
        pltpu.SemaphoreType.DMA,
        pltpu.SemaphoreType.DMA,
        pltpu.SemaphoreType.DMA,
    ],
)
def _sc_layer(h_hbm, led_hbm, agg_hbm, src_v, dst_v, pk_v, rows_v,
              h_sh, acc_sh, g0, g1, s0, s1):
    gsems = (g0, g1)
    ssems = (s0, s1)
    cid = lax.axis_index("c")
    sid = lax.axis_index("s")
    wid = cid * 16 + sid

    zv = jnp.zeros((16,), jnp.float32)

    def zbody(i, c):
        for jj in range(8):
            rows_v[0, i, pl.ds(jj * 16, 16)] = zv
        return c

    lax.fori_loop(0, CHUNK, zbody, 0)
    base = sid * RPT
    for k in range(RPT // CHUNK):
        pltpu.sync_copy(rows_v.at[0], acc_sh.at[pl.ds(base + k * CHUNK, CHUNK)])
    rem = RPT % CHUNK
    if rem:
        pltpu.sync_copy(rows_v.at[0, pl.ds(0, rem)],
                        acc_sh.at[pl.ds(base + RPT - rem, rem)])

    def g_start(u, buf):
        pltpu.make_async_copy(h_sh.at[src_v.at[u]], rows_v.at[buf],
                              gsems[buf]).start()

    def g_wait(u, buf):
        pltpu.make_async_copy(h_sh.at[src_v.at[u]], rows_v.at[buf],
                              gsems[buf]).wait()

    def s_start(u, buf):
        pltpu.async_copy(rows_v.at[buf], acc_sh.at[dst_v.at[u]],
                         ssems[buf], add=True)

    def s_wait(u, buf):
        pltpu.make_async_copy(rows_v.at[buf], acc_sh.at[dst_v.at[u]],
                              ssems[buf]).wait()

    for p in range(3):
        soff = pl.multiple_of(p * THIRD + sid * SROWS, 8)
        if p < 2:
            pltpu.sync_copy(h_hbm.at[pl.ds(soff, SROWS)],
                            h_sh.at[pl.ds(sid * SROWS, SROWS)])
        else:
            @pl.when(sid <= 13)
            def _():
                pltpu.sync_copy(h_hbm.at[pl.ds(soff, SROWS)],
                                h_sh.at[pl.ds(sid * SROWS, SROWS)])

            @pl.when(sid == 14)
            def _():
                pltpu.sync_copy(
                    h_hbm.at[pl.ds(pl.multiple_of(2 * THIRD + 14 * SROWS, 8),
                                   N - 2 * THIRD - 14 * SROWS)],
                    h_sh.at[pl.ds(14 * SROWS, N - 2 * THIRD - 14 * SROWS)])

        plsc.subcore_barrier()

        mask16 = jnp.full((16,), 0xFFFF, jnp.int32)
        sh16 = jnp.full((16,), 16, jnp.int32)

        def blk(b, c):
            boff = pl.multiple_of(b * (BCH * CHUNK), 8)
            pltpu.sync_copy(led_hbm.at[wid * 3 + p, 0,
                                       pl.ds(boff, BCH * CHUNK)], pk_v)
            for r in range(BCH):
                for q in range(CHUNK // 16):
                    w = pk_v[pl.ds(r * CHUNK + q * 16, 16)]
                    src_v[r, pl.ds(q * 16, 16)] = jnp.bitwise_and(w, mask16)
                    dst_v[r, pl.ds(q * 16, 16)] = lax.shift_right_logical(
                        w, sh16)
            g_start(0, 0)
            g_start(1, 1)
            for u in range(BCH):
                g_wait(u, u % 2)
                s_start(u, u % 2)
                s_wait(u, u % 2)
                if u + 2 < BCH:
                    g_start(u + 2, u % 2)
            return c

        lax.fori_loop(0, NBK, blk, 0)
        plsc.subcore_barrier()

    pltpu.sync_copy(acc_sh.at[pl.ds(base, RPT)],
                    agg_hbm.at[cid, pl.ds(base, RPT)])


def _mlp_body(h_ref, a_ref, batch_ref, w1_ref, b1_ref, w2_ref, b2_ref,
              out_ref, pool_ref):
    a = h_ref[...] + a_ref[0] + a_ref[1]
    t = jnp.maximum(
        jnp.dot(a, w1_ref[...], preferred_element_type=jnp.float32)
        + b1_ref[...], 0.0)
    o = jnp.maximum(
        jnp.dot(t, w2_ref[...], preferred_element_type=jnp.float32)
        + b2_ref[...], 0.0)
    out_ref[...] = o
    bt = jnp.reshape(batch_ref[...], (1, BR))
    onehot = (lax.broadcasted_iota(jnp.int32, (G, BR), 0) == bt
              ).astype(jnp.float32)
    contrib = jnp.dot(onehot, o, preferred_element_type=jnp.float32)

    @pl.when(pl.program_id(0) == 0)
    def _():
        pool_ref[...] = contrib

    @pl.when(pl.program_id(0) != 0)
    def _():
        pool_ref[...] += contrib


_mlp = pl.pallas_call(
    _mlp_body,
    grid=(N // BR,),
    in_specs=[
        pl.BlockSpec((BR, 128), lambda i: (i, 0)),
        pl.BlockSpec((2, BR, 128), lambda i: (0, i, 0)),
        pl.BlockSpec((1, 1, BR), lambda i: (i, 0, 0)),
        pl.BlockSpec((128, 128), lambda i: (0, 0)),
        pl.BlockSpec((1, 128), lambda i: (0, 0)),
        pl.BlockSpec((128, 128), lambda i: (0, 0)),
        pl.BlockSpec((1, 128), lambda i: (0, 0)),
    ],
    out_specs=[
        pl.BlockSpec((BR, 128), lambda i: (i, 0)),
        pl.BlockSpec((G, 128), lambda i: (0, 0)),
    ],
    out_shape=[
        jax.ShapeDtypeStruct((N, 128), jnp.float32),
        jax.ShapeDtypeStruct((G, 128), jnp.float32),
    ],
    compiler_params=pltpu.CompilerParams(
        dimension_semantics=("arbitrary",)),
)


def kernel(x, edge_index, batch, W1_0, b1_0, W2_0, b2_0, W1_1, b1_1, W2_1,
           b2_1, W1_2, b1_2, W2_2, b2_2):
    src = edge_index[0].astype(jnp.int32)
    dst = edge_index[1].astype(jnp.int32)
    pad = NW * EPW - E
    pad_src = (jnp.arange(pad, dtype=jnp.int32) * N) // pad
    src_p = jnp.concatenate([src, pad_src]).reshape(NW, EPW)
    dst_p = jnp.concatenate([dst, jnp.full((pad,), DUMMY, jnp.int32)]
                            ).reshape(NW, EPW)
    batch3 = batch.astype(jnp.int32).reshape(N // BR, 1, BR)

    led4 = _sc_partition(src_p, dst_p)

    layers = [(W1_0, b1_0, W2_0, b2_0), (W1_1, b1_1, W2_1, b2_1),
              (W1_2, b1_2, W2_2, b2_2)]
    h = x
    hs, pools = [], []
    for (W1, b1, W2, b2) in layers:
        agg = _sc_layer(h, led4)
        h, pool = _mlp(h, agg, batch3, W1, b1.reshape(1, 128), W2,
                       b2.reshape(1, 128))
        hs.append(h)
        pools.append(pool)
    graph_emb = jnp.concatenate(pools, axis=1)
    node_emb = jnp.concatenate(hs, axis=1)
    return (graph_emb, node_emb)
